# baseline (speedup 1.0000x reference)
import functools

import jax
import jax.numpy as jnp
from jax.experimental import pallas as pl
from jax.experimental.pallas import tpu as pltpu


BB = 16


def _mish(x):
    e = jnp.exp(jnp.minimum(x, 20.0))
    n = (1.0 + e) * (1.0 + e)
    t = (n - 1.0) / (n + 1.0)
    return x * t


def _bmm(w_t, x3):
    bb = x3.shape[0]
    return jnp.concatenate(
        [jnp.dot(w_t, x3[b], preferred_element_type=jnp.float32)[None]
         for b in range(bb)], axis=0)


def _gnn_kernel(n_layers, h_dim,
                ei_ref, nf_ref, ef_ref,
                wn_ref, we_ref,
                proj_ref, mw1c_ref, mw2_ref, mb2_ref,
                uw1b_ref, ub1_ref, uw2_ref, ub2_ref,
                lng_ref, lnb_ref,
                ow1_ref, ob1_ref, ow2_ref, ob2_ref,
                emb_ref, board_ref,
                s_ref, d_ref, dt_ref):
    H = h_dim
    N = nf_ref.shape[2]
    E = ef_ref.shape[2]
    bb = nf_ref.shape[0]
    bf16 = jnp.bfloat16

    @pl.when(pl.program_id(0) == 0)
    def _build_onehots():
        src = ei_ref[0:1, :]
        dst = ei_ref[1:2, :]
        iota_n = jax.lax.broadcasted_iota(jnp.int32, (N, E), 0)
        d = (iota_n == dst).astype(bf16)
        s_ref[...] = (iota_n == src).astype(bf16)
        d_ref[...] = d
        dt_ref[...] = jnp.transpose(d)

    S = s_ref[...]
    D = d_ref[...]
    Dt = dt_ref[...]

    h3 = _mish(_bmm(wn_ref[...], nf_ref[...]))
    e3 = _bmm(we_ref[...], ef_ref[...]).astype(bf16)

    epre_all = _bmm(mw1c_ref[...], e3).astype(bf16)

    for l in range(n_layers):
        p = _bmm(proj_ref[l], h3.astype(bf16))
        hs2 = p[:, :H].astype(bf16).reshape(bb * H, N)
        hd2 = p[:, H:2 * H].astype(bf16).reshape(bb * H, N)
        uh = p[:, 2 * H:]
        msrc = jnp.dot(hs2, S, preferred_element_type=jnp.float32)
        mdst = jnp.dot(hd2, D, preferred_element_type=jnp.float32)
        m1 = (msrc.reshape(bb, H, E).astype(bf16)
              + mdst.reshape(bb, H, E).astype(bf16)
              + epre_all[:, l * H:(l + 1) * H])
        m3 = _mish(m1)
        m2 = (_bmm(mw2_ref[l], m3).astype(bf16)
              + mb2_ref[l][None, :, :].astype(bf16))
        m3 = _mish(m2)
        agg = jnp.dot(m3.reshape(bb * H, E), Dt,
                      preferred_element_type=jnp.float32).reshape(bb, H, N)
        u3 = _mish(uh + _bmm(uw1b_ref[l], agg.astype(bf16))
                   + ub1_ref[l][None, :, :])
        u3 = _bmm(uw2_ref[l], u3.astype(bf16)) + ub2_ref[l][None, :, :]
        r3 = h3 + u3
        mu = jnp.mean(r3, axis=1, keepdims=True)
        var = jnp.mean((r3 - mu) * (r3 - mu), axis=1, keepdims=True)
        h3 = ((r3 - mu) / jnp.sqrt(var + 1e-5) * lng_ref[l][None, :, :]
              + lnb_ref[l][None, :, :])

    emb_ref[...] = h3

    meanp = jnp.mean(h3, axis=2)
    maxp = jnp.max(h3, axis=2)
    pooled_t = jnp.concatenate(
        [jnp.transpose(meanp), jnp.transpose(maxp)], axis=0)
    o1 = _mish(jnp.dot(ow1_ref[...], pooled_t,
                       preferred_element_type=jnp.float32) + ob1_ref[...])
    o2 = (jnp.dot(ow2_ref[...], o1, preferred_element_type=jnp.float32)
          + ob2_ref[...])
    board_ref[...] = jnp.transpose(o2)


def kernel(node_features, edge_index, edge_features, W_node, b_node, W_edge,
           b_edge, msg_W1, msg_b1, msg_W2, msg_b2, upd_W1, upd_b1, upd_W2,
           upd_b2, ln_g, ln_b, out_W1, out_b1, out_W2, out_b2):
    B, N, DN = node_features.shape
    E = edge_features.shape[1]
    DE = edge_features.shape[2]
    L, H3, H = msg_W1.shape
    H = msg_W2.shape[1]
    OUT = out_W1.shape[1]
    assert B % BB == 0
    grid = B // BB

    f32 = jnp.float32
    bf16 = jnp.bfloat16
    nf_t = jnp.concatenate(
        [jnp.transpose(node_features, (0, 2, 1)),
         jnp.ones((B, 1, N), f32)], axis=1).astype(bf16)
    ef_t = jnp.concatenate(
        [jnp.transpose(edge_features, (0, 2, 1)),
         jnp.ones((B, 1, E), f32)], axis=1).astype(bf16)
    wn_t = jnp.concatenate(
        [jnp.transpose(W_node), b_node[:, None]], axis=1).astype(bf16)
    we_t = jnp.concatenate(
        [jnp.concatenate([jnp.transpose(W_edge), b_edge[:, None]], axis=1),
         jnp.concatenate([jnp.zeros((1, DE), f32),
                          jnp.ones((1, 1), f32)], axis=1)],
        axis=0).astype(bf16)
    proj_t = jnp.concatenate(
        [jnp.transpose(msg_W1[:, :H, :], (0, 2, 1)),
         jnp.transpose(msg_W1[:, H:2 * H, :], (0, 2, 1)),
         jnp.transpose(upd_W1[:, :H, :], (0, 2, 1))],
        axis=1).astype(bf16)
    mw1c_t = jnp.concatenate(
        [jnp.transpose(msg_W1[:, 2 * H:, :], (0, 2, 1)).reshape(L * H, H),
         msg_b1.reshape(L * H, 1)], axis=1).astype(bf16)
    mw2_t = jnp.transpose(msg_W2, (0, 2, 1)).astype(bf16)
    uw1b_t = jnp.transpose(upd_W1[:, H:, :], (0, 2, 1)).astype(bf16)
    uw2_t = jnp.transpose(upd_W2, (0, 2, 1)).astype(bf16)
    ow1_t = jnp.transpose(out_W1)
    ow2_t = jnp.transpose(out_W2)
    mb2 = msg_b2.reshape(L, H, 1)
    ub1 = upd_b1.reshape(L, H, 1)
    ub2 = upd_b2.reshape(L, H, 1)
    lng = ln_g.reshape(L, H, 1)
    lnb = ln_b.reshape(L, H, 1)
    ob1 = out_b1.reshape(OUT, 1)
    ob2 = out_b2.reshape(OUT, 1)

    def full(shape):
        r = len(shape)
        return pl.BlockSpec(shape, lambda i, _r=r: (0,) * _r)

    in_specs = [
        pl.BlockSpec((2, E), lambda i: (0, 0)),
        pl.BlockSpec((BB, DN + 1, N), lambda i: (i, 0, 0)),
        pl.BlockSpec((BB, DE + 1, E), lambda i: (i, 0, 0)),
        full((H, DN + 1)), full((H + 1, DE + 1)),
        full((L, 3 * H, H)), full((L * H, H + 1)),
        full((L, H, H)), full((L, H, 1)),
        full((L, H, H)), full((L, H, 1)),
        full((L, H, H)), full((L, H, 1)),
        full((L, H, 1)), full((L, H, 1)),
        full((OUT, 2 * H)), full((OUT, 1)),
        full((OUT, OUT)), full((OUT, 1)),
    ]
    out_specs = [
        pl.BlockSpec((BB, H, N), lambda i: (i, 0, 0)),
        pl.BlockSpec((BB, OUT), lambda i: (i, 0)),
    ]
    out_shape = [
        jax.ShapeDtypeStruct((B, H, N), f32),
        jax.ShapeDtypeStruct((B, OUT), f32),
    ]

    node_emb_t, board_emb = pl.pallas_call(
        functools.partial(_gnn_kernel, L, H),
        grid=(grid,),
        in_specs=in_specs,
        out_specs=out_specs,
        out_shape=out_shape,
        scratch_shapes=[pltpu.VMEM((N, E), bf16),
                        pltpu.VMEM((N, E), bf16),
                        pltpu.VMEM((E, N), bf16)],
    )(edge_index, nf_t, ef_t, wn_t, we_t,
      proj_t, mw1c_t, mw2_t, mb2, uw1b_t, ub1, uw2_t, ub2,
      lng, lnb, ow1_t, ob1, ow2_t, ob2)

    node_emb = jnp.transpose(node_emb_t, (0, 2, 1))
    return (board_emb, node_emb)


# speedup vs baseline: 1.0027x; 1.0027x over previous
import functools

import jax
import jax.numpy as jnp
from jax.experimental import pallas as pl
from jax.experimental.pallas import tpu as pltpu


BB = 16


def _mish(x):
    e = jnp.exp(jnp.minimum(x, 20.0))
    n = (1.0 + e) * (1.0 + e)
    t = (n - 1.0) / (n + 1.0)
    return x * t


def _bmm(w_t, x3):
    bb = x3.shape[0]
    return jnp.concatenate(
        [jnp.dot(w_t, x3[b], preferred_element_type=jnp.float32)[None]
         for b in range(bb)], axis=0)


def _gnn_kernel(n_layers, h_dim,
                ei_ref, nf_ref, ef_ref,
                wn_ref, we_ref,
                proj_ref, mw1c_ref, mw2_ref, mb2_ref,
                uw1b_ref, ub1_ref, uw2_ref, ub2_ref,
                lng_ref, lnb_ref,
                ow1_ref, ob1_ref, ow2_ref, ob2_ref,
                emb_ref, board_ref,
                s_ref, d_ref, dt_ref):
    H = h_dim
    N = nf_ref.shape[2]
    E = ef_ref.shape[2]
    bb = nf_ref.shape[0]
    bf16 = jnp.bfloat16

    @pl.when(pl.program_id(0) == 0)
    def _build_onehots():
        src = ei_ref[0:1, :]
        dst = ei_ref[1:2, :]
        iota_n = jax.lax.broadcasted_iota(jnp.int32, (N, E), 0)
        d = (iota_n == dst).astype(bf16)
        s_ref[...] = (iota_n == src).astype(bf16)
        d_ref[...] = d
        dt_ref[...] = jnp.transpose(d)

    S = s_ref[...]
    D = d_ref[...]
    Dt = dt_ref[...]

    h3 = _mish(_bmm(wn_ref[...], nf_ref[...]))
    e3 = _bmm(we_ref[...], ef_ref[...]).astype(bf16)

    epre_all = _bmm(mw1c_ref[...], e3).astype(bf16)

    for l in range(n_layers):
        p = _bmm(proj_ref[l], h3.astype(bf16))
        hs2 = p[:, :H].astype(bf16).reshape(bb * H, N)
        hd2 = p[:, H:2 * H].astype(bf16).reshape(bb * H, N)
        uh = p[:, 2 * H:]
        msrc = jnp.dot(hs2, S, preferred_element_type=jnp.float32)
        mdst = jnp.dot(hd2, D, preferred_element_type=jnp.float32)
        m1 = (msrc.reshape(bb, H, E).astype(bf16)
              + mdst.reshape(bb, H, E).astype(bf16)
              + epre_all[:, l * H:(l + 1) * H])
        m3 = _mish(m1)
        m2 = (_bmm(mw2_ref[l], m3).astype(bf16)
              + mb2_ref[l][None, :, :].astype(bf16))
        m3 = _mish(m2)
        agg = jnp.dot(m3.reshape(bb * H, E), Dt,
                      preferred_element_type=jnp.float32).reshape(bb, H, N)
        u3 = _mish(uh + _bmm(uw1b_ref[l], agg.astype(bf16))
                   + ub1_ref[l][None, :, :])
        u3 = _bmm(uw2_ref[l], u3.astype(bf16)) + ub2_ref[l][None, :, :]
        r3 = h3 + u3
        mu = jnp.mean(r3, axis=1, keepdims=True)
        var = jnp.mean((r3 - mu) * (r3 - mu), axis=1, keepdims=True)
        h3 = ((r3 - mu) / jnp.sqrt(var + 1e-5) * lng_ref[l][None, :, :]
              + lnb_ref[l][None, :, :])

    emb_ref[...] = h3

    meanp = jnp.mean(h3, axis=2)
    maxp = jnp.max(h3, axis=2)
    pooled_t = jnp.concatenate(
        [jnp.transpose(meanp), jnp.transpose(maxp)], axis=0)
    o1 = _mish(jnp.dot(ow1_ref[...], pooled_t,
                       preferred_element_type=jnp.float32) + ob1_ref[...])
    o2 = (jnp.dot(ow2_ref[...], o1, preferred_element_type=jnp.float32)
          + ob2_ref[...])
    board_ref[...] = jnp.transpose(o2)


def kernel(node_features, edge_index, edge_features, W_node, b_node, W_edge,
           b_edge, msg_W1, msg_b1, msg_W2, msg_b2, upd_W1, upd_b1, upd_W2,
           upd_b2, ln_g, ln_b, out_W1, out_b1, out_W2, out_b2):
    B, N, DN = node_features.shape
    E = edge_features.shape[1]
    DE = edge_features.shape[2]
    L, H3, H = msg_W1.shape
    H = msg_W2.shape[1]
    OUT = out_W1.shape[1]
    assert B % BB == 0
    grid = B // BB

    f32 = jnp.float32
    bf16 = jnp.bfloat16
    nf_t = jnp.concatenate(
        [jnp.transpose(node_features, (0, 2, 1)),
         jnp.ones((B, 1, N), f32)], axis=1).astype(bf16)
    ef_t = jnp.concatenate(
        [jnp.transpose(edge_features, (0, 2, 1)),
         jnp.ones((B, 1, E), f32)], axis=1).astype(bf16)
    wn_t = jnp.concatenate(
        [jnp.transpose(W_node), b_node[:, None]], axis=1).astype(bf16)
    HE = (H + 16) // 16 * 16
    ones_zero_rows = jnp.concatenate(
        [jnp.zeros((HE - H, DE), f32),
         jnp.concatenate([jnp.ones((1, 1), f32),
                          jnp.zeros((HE - H - 1, 1), f32)], axis=0)], axis=1)
    we_t = jnp.concatenate(
        [jnp.concatenate([jnp.transpose(W_edge), b_edge[:, None]], axis=1),
         ones_zero_rows], axis=0).astype(bf16)
    proj_t = jnp.concatenate(
        [jnp.transpose(msg_W1[:, :H, :], (0, 2, 1)),
         jnp.transpose(msg_W1[:, H:2 * H, :], (0, 2, 1)),
         jnp.transpose(upd_W1[:, :H, :], (0, 2, 1))],
        axis=1).astype(bf16)
    mw1c_t = jnp.concatenate(
        [jnp.transpose(msg_W1[:, 2 * H:, :], (0, 2, 1)).reshape(L * H, H),
         msg_b1.reshape(L * H, 1),
         jnp.zeros((L * H, HE - H - 1), f32)], axis=1).astype(bf16)
    mw2_t = jnp.transpose(msg_W2, (0, 2, 1)).astype(bf16)
    uw1b_t = jnp.transpose(upd_W1[:, H:, :], (0, 2, 1)).astype(bf16)
    uw2_t = jnp.transpose(upd_W2, (0, 2, 1)).astype(bf16)
    ow1_t = jnp.transpose(out_W1)
    ow2_t = jnp.transpose(out_W2)
    mb2 = msg_b2.reshape(L, H, 1)
    ub1 = upd_b1.reshape(L, H, 1)
    ub2 = upd_b2.reshape(L, H, 1)
    lng = ln_g.reshape(L, H, 1)
    lnb = ln_b.reshape(L, H, 1)
    ob1 = out_b1.reshape(OUT, 1)
    ob2 = out_b2.reshape(OUT, 1)

    def full(shape):
        r = len(shape)
        return pl.BlockSpec(shape, lambda i, _r=r: (0,) * _r)

    in_specs = [
        pl.BlockSpec((2, E), lambda i: (0, 0)),
        pl.BlockSpec((BB, DN + 1, N), lambda i: (i, 0, 0)),
        pl.BlockSpec((BB, DE + 1, E), lambda i: (i, 0, 0)),
        full((H, DN + 1)), full(((H + 16) // 16 * 16, DE + 1)),
        full((L, 3 * H, H)), full((L * H, (H + 16) // 16 * 16)),
        full((L, H, H)), full((L, H, 1)),
        full((L, H, H)), full((L, H, 1)),
        full((L, H, H)), full((L, H, 1)),
        full((L, H, 1)), full((L, H, 1)),
        full((OUT, 2 * H)), full((OUT, 1)),
        full((OUT, OUT)), full((OUT, 1)),
    ]
    out_specs = [
        pl.BlockSpec((BB, H, N), lambda i: (i, 0, 0)),
        pl.BlockSpec((BB, OUT), lambda i: (i, 0)),
    ]
    out_shape = [
        jax.ShapeDtypeStruct((B, H, N), f32),
        jax.ShapeDtypeStruct((B, OUT), f32),
    ]

    node_emb_t, board_emb = pl.pallas_call(
        functools.partial(_gnn_kernel, L, H),
        grid=(grid,),
        in_specs=in_specs,
        out_specs=out_specs,
        out_shape=out_shape,
        scratch_shapes=[pltpu.VMEM((N, E), bf16),
                        pltpu.VMEM((N, E), bf16),
                        pltpu.VMEM((E, N), bf16)],
    )(edge_index, nf_t, ef_t, wn_t, we_t,
      proj_t, mw1c_t, mw2_t, mb2, uw1b_t, ub1, uw2_t, ub2,
      lng, lnb, ow1_t, ob1, ow2_t, ob2)

    node_emb = jnp.transpose(node_emb_t, (0, 2, 1))
    return (board_emb, node_emb)


# speedup vs baseline: 1.0259x; 1.0232x over previous
import functools

import jax
import jax.numpy as jnp
from jax.experimental import pallas as pl
from jax.experimental.pallas import tpu as pltpu


BB = 16


def _mish(x):
    e = jnp.exp(jnp.minimum(x, 20.0))
    n = (1.0 + e) * (1.0 + e)
    t = (n - 1.0) / (n + 1.0)
    return x * t


def _bmm(w_t, x3):
    bb = x3.shape[0]
    return jnp.concatenate(
        [jnp.dot(w_t, x3[b], preferred_element_type=jnp.float32)[None]
         for b in range(bb)], axis=0)


def _gnn_kernel(n_layers, h_dim,
                ei_ref, nf_ref, ef_ref,
                wn_ref, bn_ref, we_ref, be_ref,
                proj_ref, mw1c_ref, mb1_ref, mw2_ref, mb2_ref,
                uw1b_ref, ub1_ref, uw2_ref, ub2_ref,
                lng_ref, lnb_ref,
                ow1_ref, ob1_ref, ow2_ref, ob2_ref,
                emb_ref, board_ref,
                s_ref, d_ref, dt_ref):
    H = h_dim
    N = nf_ref.shape[2]
    E = ef_ref.shape[2]
    bb = nf_ref.shape[0]
    bf16 = jnp.bfloat16

    @pl.when(pl.program_id(0) == 0)
    def _build_onehots():
        src = ei_ref[0:1, :]
        dst = ei_ref[1:2, :]
        iota_n = jax.lax.broadcasted_iota(jnp.int32, (N, E), 0)
        d = (iota_n == dst).astype(bf16)
        s_ref[...] = (iota_n == src).astype(bf16)
        d_ref[...] = d
        dt_ref[...] = jnp.transpose(d)

    S = s_ref[...]
    D = d_ref[...]
    Dt = dt_ref[...]

    h3 = _mish(_bmm(wn_ref[...], nf_ref[...]) + bn_ref[...][None])
    e3 = (_bmm(we_ref[...], ef_ref[...])
          + be_ref[...][None]).astype(bf16)

    epre_all = _bmm(mw1c_ref[...], e3).astype(bf16)

    for l in range(n_layers):
        p = _bmm(proj_ref[l], h3.astype(bf16))
        hs2 = p[:, :H].astype(bf16).reshape(bb * H, N)
        hd2 = p[:, H:2 * H].astype(bf16).reshape(bb * H, N)
        uh = p[:, 2 * H:]
        msrc = jnp.dot(hs2, S, preferred_element_type=jnp.float32)
        mdst = jnp.dot(hd2, D, preferred_element_type=jnp.float32)
        m1 = (msrc.reshape(bb, H, E).astype(bf16)
              + mdst.reshape(bb, H, E).astype(bf16)
              + epre_all[:, l * H:(l + 1) * H]
              + mb1_ref[l][None, :, :].astype(bf16))
        m3 = _mish(m1)
        m2 = (_bmm(mw2_ref[l], m3).astype(bf16)
              + mb2_ref[l][None, :, :].astype(bf16))
        m3 = _mish(m2)
        agg = jnp.dot(m3.reshape(bb * H, E), Dt,
                      preferred_element_type=jnp.float32).reshape(bb, H, N)
        u3 = _mish(uh + _bmm(uw1b_ref[l], agg.astype(bf16))
                   + ub1_ref[l][None, :, :])
        u3 = _bmm(uw2_ref[l], u3.astype(bf16)) + ub2_ref[l][None, :, :]
        r3 = h3 + u3
        mu = jnp.mean(r3, axis=1, keepdims=True)
        var = jnp.mean((r3 - mu) * (r3 - mu), axis=1, keepdims=True)
        h3 = ((r3 - mu) / jnp.sqrt(var + 1e-5) * lng_ref[l][None, :, :]
              + lnb_ref[l][None, :, :])

    emb_ref[...] = h3

    meanp = jnp.mean(h3, axis=2)
    maxp = jnp.max(h3, axis=2)
    pooled_t = jnp.concatenate(
        [jnp.transpose(meanp), jnp.transpose(maxp)], axis=0)
    o1 = _mish(jnp.dot(ow1_ref[...], pooled_t,
                       preferred_element_type=jnp.float32) + ob1_ref[...])
    o2 = (jnp.dot(ow2_ref[...], o1, preferred_element_type=jnp.float32)
          + ob2_ref[...])
    board_ref[...] = jnp.transpose(o2)


def kernel(node_features, edge_index, edge_features, W_node, b_node, W_edge,
           b_edge, msg_W1, msg_b1, msg_W2, msg_b2, upd_W1, upd_b1, upd_W2,
           upd_b2, ln_g, ln_b, out_W1, out_b1, out_W2, out_b2):
    B, N, DN = node_features.shape
    E = edge_features.shape[1]
    DE = edge_features.shape[2]
    L, H3, H = msg_W1.shape
    H = msg_W2.shape[1]
    OUT = out_W1.shape[1]
    assert B % BB == 0
    grid = B // BB

    f32 = jnp.float32
    bf16 = jnp.bfloat16
    nf_t = jnp.transpose(node_features, (0, 2, 1)).astype(bf16)
    ef_t = jnp.transpose(edge_features, (0, 2, 1)).astype(bf16)
    wn_t = jnp.transpose(W_node).astype(bf16)
    we_t = jnp.transpose(W_edge).astype(bf16)
    proj_t = jnp.concatenate(
        [jnp.transpose(msg_W1[:, :H, :], (0, 2, 1)),
         jnp.transpose(msg_W1[:, H:2 * H, :], (0, 2, 1)),
         jnp.transpose(upd_W1[:, :H, :], (0, 2, 1))],
        axis=1).astype(bf16)
    mw1c_t = jnp.transpose(msg_W1[:, 2 * H:, :],
                           (0, 2, 1)).reshape(L * H, H).astype(bf16)
    mw2_t = jnp.transpose(msg_W2, (0, 2, 1)).astype(bf16)
    uw1b_t = jnp.transpose(upd_W1[:, H:, :], (0, 2, 1)).astype(bf16)
    uw2_t = jnp.transpose(upd_W2, (0, 2, 1)).astype(bf16)
    ow1_t = jnp.transpose(out_W1)
    ow2_t = jnp.transpose(out_W2)
    bn = b_node.reshape(H, 1)
    be = b_edge.reshape(H, 1)
    mb1 = msg_b1.reshape(L, H, 1)
    mb2 = msg_b2.reshape(L, H, 1)
    ub1 = upd_b1.reshape(L, H, 1)
    ub2 = upd_b2.reshape(L, H, 1)
    lng = ln_g.reshape(L, H, 1)
    lnb = ln_b.reshape(L, H, 1)
    ob1 = out_b1.reshape(OUT, 1)
    ob2 = out_b2.reshape(OUT, 1)

    def full(shape):
        r = len(shape)
        return pl.BlockSpec(shape, lambda i, _r=r: (0,) * _r)

    in_specs = [
        pl.BlockSpec((2, E), lambda i: (0, 0)),
        pl.BlockSpec((BB, DN, N), lambda i: (i, 0, 0)),
        pl.BlockSpec((BB, DE, E), lambda i: (i, 0, 0)),
        full((H, DN)), full((H, 1)), full((H, DE)), full((H, 1)),
        full((L, 3 * H, H)), full((L * H, H)), full((L, H, 1)),
        full((L, H, H)), full((L, H, 1)),
        full((L, H, H)), full((L, H, 1)),
        full((L, H, H)), full((L, H, 1)),
        full((L, H, 1)), full((L, H, 1)),
        full((OUT, 2 * H)), full((OUT, 1)),
        full((OUT, OUT)), full((OUT, 1)),
    ]
    out_specs = [
        pl.BlockSpec((BB, H, N), lambda i: (i, 0, 0)),
        pl.BlockSpec((BB, OUT), lambda i: (i, 0)),
    ]
    out_shape = [
        jax.ShapeDtypeStruct((B, H, N), f32),
        jax.ShapeDtypeStruct((B, OUT), f32),
    ]

    node_emb_t, board_emb = pl.pallas_call(
        functools.partial(_gnn_kernel, L, H),
        grid=(grid,),
        in_specs=in_specs,
        out_specs=out_specs,
        out_shape=out_shape,
        scratch_shapes=[pltpu.VMEM((N, E), bf16),
                        pltpu.VMEM((N, E), bf16),
                        pltpu.VMEM((E, N), bf16)],
    )(edge_index, nf_t, ef_t, wn_t, bn, we_t, be,
      proj_t, mw1c_t, mb1, mw2_t, mb2, uw1b_t, ub1, uw2_t, ub2,
      lng, lnb, ow1_t, ob1, ow2_t, ob2)

    node_emb = jnp.transpose(node_emb_t, (0, 2, 1))
    return (board_emb, node_emb)
